# baseline (speedup 1.0000x reference)
import jax
import jax.numpy as jnp
from jax.experimental import pallas as pl

H = 0.1


def _tv_norm(X, eps=0.001):
    X = X - jnp.mean(X, axis=1, keepdims=True)
    X = X / jnp.sqrt(jnp.sum(X ** 2, axis=1, keepdims=True) + eps)
    return X


def kernel(xn, xe, edge_index, KNopen, KEopen, KNclose, KEclose, KN, KE, KD):
    iInd = edge_index[0]
    jInd = edge_index[1]
    n_nodes = xn.shape[1]
    xn = KNopen @ xn
    xe = KEopen @ xe
    for i in range(KN.shape[0]):
        g = xn[:, iInd] - xn[:, jInd]
        Ai = _tv_norm(KN[i] @ g)
        xe = xe + H * jax.nn.relu(Ai)
        div = jnp.zeros((xe.shape[0], n_nodes), dtype=xe.dtype)
        div = div.at[:, iInd].add(xe)
        div = div.at[:, jInd].add(-xe)
        Bi = jax.nn.relu(_tv_norm(KE[i] @ div))
        Ci = jax.nn.relu(_tv_norm(KD[i] @ xn))
        xn = xn - H * (Bi + Ci)
    xn = KNclose @ xn
    xe = KEclose @ xe
    return (xn, xe)


# speedup vs baseline: 3.1166x; 3.1166x over previous
import functools

import jax
import jax.numpy as jnp
from jax import lax
from jax.experimental import pallas as pl
from jax.experimental.pallas import tpu as pltpu
from jax.experimental.pallas import tpu_sc as plsc

H = 0.1
EPS = 1e-3
F = 128
NW = 32
LANES = 16
Nwhite = None
C = 80
BE = 2560


def _sc_mesh():
    return plsc.VectorSubcoreMesh(core_axis_name="c", subcore_axis_name="s")


def _make_pass_a(N, E):
    EW = E // NW
    NCH = EW // C
    nv = F // LANES

    @functools.partial(
        pl.kernel,
        mesh=_sc_mesh(),
        out_type=jax.ShapeDtypeStruct((NW, 2, F), jnp.float32),
        scratch_types=[
            pltpu.VMEM((C,), jnp.int32),
            pltpu.VMEM((C,), jnp.int32),
            pltpu.VMEM((C, F), jnp.float32),
            pltpu.VMEM((C, F), jnp.float32),
            pltpu.VMEM((2, F), jnp.float32),
            pltpu.SemaphoreType.DMA,
            pltpu.SemaphoreType.DMA,
        ],
    )
    def pass_a(ii, jj, yt, stats, idx_i, idx_j, rows_i, rows_j, st, sem1, sem2):
        wid = lax.axis_index("c") * 16 + lax.axis_index("s")
        base = wid * EW

        def chunk(c, carry):
            off = base + c * C
            pltpu.sync_copy(ii.at[pl.ds(off, C)], idx_i)
            pltpu.sync_copy(jj.at[pl.ds(off, C)], idx_j)
            cp1 = pltpu.async_copy(yt.at[idx_i], rows_i, sem1)
            cp2 = pltpu.async_copy(yt.at[idx_j], rows_j, sem2)
            cp1.wait()
            cp2.wait()

            def edge(e, acc):
                out = list(acc)
                for v in range(nv):
                    sl = pl.ds(v * LANES, LANES)
                    d = rows_i[e, sl] - rows_j[e, sl]
                    out[v] = acc[v] + d
                    out[nv + v] = acc[nv + v] + d * d
                return tuple(out)

            return lax.fori_loop(0, C, edge, carry)

        zero = jnp.zeros((LANES,), jnp.float32)
        acc = lax.fori_loop(0, NCH, chunk, tuple(zero for _ in range(2 * nv)))
        for v in range(nv):
            st[0, pl.ds(v * LANES, LANES)] = acc[v]
            st[1, pl.ds(v * LANES, LANES)] = acc[nv + v]
        pltpu.sync_copy(st, stats.at[wid])

    return pass_a


def _make_pass_b(N, E):
    EW = E // NW
    NCH = EW // C
    nv = F // LANES
    RPT = 1000

    @functools.partial(
        pl.kernel,
        mesh=_sc_mesh(),
        out_type=(
            jax.ShapeDtypeStruct((E, F), jnp.float32),
            jax.ShapeDtypeStruct((2, N, F), jnp.float32),
        ),
        scratch_types=[
            pltpu.VMEM((C,), jnp.int32),
            pltpu.VMEM((C,), jnp.int32),
            pltpu.VMEM((C, F), jnp.float32),
            pltpu.VMEM((C, F), jnp.float32),
            pltpu.VMEM((C, F), jnp.float32),
            pltpu.VMEM((C, F), jnp.float32),
            pltpu.VMEM((2, F), jnp.float32),
            pltpu.VMEM_SHARED((N, F), jnp.float32),
            pltpu.SemaphoreType.DMA,
            pltpu.SemaphoreType.DMA,
        ],
    )
    def pass_b(ii, jj, yt, xet, mi, zeros, xet_out, divp,
               idx_i, idx_j, rows_i, rows_j, xe_buf, neg_buf, mi_buf,
               shared, sem1, sem2):
        cid = lax.axis_index("c")
        sid = lax.axis_index("s")
        wid = cid * 16 + sid
        base = wid * EW

        pltpu.sync_copy(mi, mi_buf)

        @pl.when(sid == 0)
        def _():
            pltpu.sync_copy(zeros, shared)

        plsc.subcore_barrier()

        mean_vs = [mi_buf[0, pl.ds(v * LANES, LANES)] for v in range(nv)]
        inv_vs = [mi_buf[1, pl.ds(v * LANES, LANES)] for v in range(nv)]

        def chunk(c, carry):
            off = base + c * C
            pltpu.sync_copy(ii.at[pl.ds(off, C)], idx_i)
            pltpu.sync_copy(jj.at[pl.ds(off, C)], idx_j)
            cp1 = pltpu.async_copy(yt.at[idx_i], rows_i, sem1)
            cp2 = pltpu.async_copy(yt.at[idx_j], rows_j, sem2)
            pltpu.sync_copy(xet.at[pl.ds(off, C)], xe_buf)
            cp1.wait()
            cp2.wait()

            def edge(e, cc):
                for v in range(nv):
                    sl = pl.ds(v * LANES, LANES)
                    r = jnp.maximum(
                        (rows_i[e, sl] - rows_j[e, sl] - mean_vs[v]) * inv_vs[v],
                        0.0)
                    val = xe_buf[e, sl] + H * r
                    xe_buf[e, sl] = val
                    neg_buf[e, sl] = -val
                return cc

            lax.fori_loop(0, C, edge, 0)
            pltpu.sync_copy(xe_buf, xet_out.at[pl.ds(off, C)])
            pltpu.sync_copy(xe_buf, shared.at[idx_i], add=True)
            pltpu.sync_copy(neg_buf, shared.at[idx_j], add=True)
            return carry

        lax.fori_loop(0, NCH, chunk, 0)
        plsc.subcore_barrier()

        @pl.when(sid < N // RPT)
        def _():
            r0 = sid * RPT
            pltpu.sync_copy(shared.at[pl.ds(r0, RPT)],
                            divp.at[cid, pl.ds(r0, RPT)])

    return pass_b


def _tvn_rows(X):
    Xc = X - jnp.mean(X, axis=1, keepdims=True)
    return Xc / jnp.sqrt(jnp.sum(Xc * Xc, axis=1, keepdims=True) + EPS)


def _dot(a, b, dims):
    return lax.dot_general(a, b, (dims, ((), ())),
                           preferred_element_type=jnp.float32)


def _open_xn_body(kno, xn, kn0, xn1_out, yt0_out):
    xn1 = _dot(kno[...], xn[...], ((1,), (0,)))
    xn1_out[...] = xn1
    yt0_out[...] = _dot(xn1, kn0[...], ((0,), (1,)))


def _layer_body(divp, xn, ke, kd, knn, xn_out, yt_out):
    P = divp[0] + divp[1]
    Bi = jnp.maximum(_tvn_rows(_dot(ke[...], P, ((1,), (1,)))), 0.0)
    Ci = jnp.maximum(_tvn_rows(_dot(kd[...], xn[...], ((1,), (0,)))), 0.0)
    xn_new = xn[...] - H * (Bi + Ci)
    xn_out[...] = xn_new
    yt_out[...] = _dot(xn_new, knn[...], ((0,), (1,)))


def _last_body(divp, xn, ke, kd, knc, xn_out):
    P = divp[0] + divp[1]
    Bi = jnp.maximum(_tvn_rows(_dot(ke[...], P, ((1,), (1,)))), 0.0)
    Ci = jnp.maximum(_tvn_rows(_dot(kd[...], xn[...], ((1,), (0,)))), 0.0)
    xn_new = xn[...] - H * (Bi + Ci)
    xn_out[...] = _dot(knc[...], xn_new, ((1,), (0,)))


def _make_fin(E):
    def _fin_body(stats, out):
        T = jnp.sum(stats[...], axis=0)
        m = T[0:1] / float(E)
        q = T[1:2]
        inv = lax.rsqrt(q - float(E) * m * m + EPS)
        out[...] = jnp.concatenate([m, inv], axis=0)
    return _fin_body


def _xe_open_body(xe, keo, out):
    out[...] = _dot(xe[...], keo[...], ((0,), (1,)))


def _xe_close_body(xet, kec, out):
    out[...] = _dot(kec[...], xet[...], ((1,), (1,)))


def kernel(xn, xe, edge_index, KNopen, KEopen, KNclose, KEclose, KN, KE, KD):
    N = xn.shape[1]
    E = xe.shape[1]
    FE = xe.shape[0]
    nL = KN.shape[0]
    ii = edge_index[0]
    jj = edge_index[1]
    zeros = jnp.zeros((N, F), jnp.float32)

    pass_a = _make_pass_a(N, E)
    pass_b = _make_pass_b(N, E)

    open_xn = pl.pallas_call(
        _open_xn_body,
        out_shape=(jax.ShapeDtypeStruct((F, N), jnp.float32),
                   jax.ShapeDtypeStruct((N, F), jnp.float32)))
    layer_tc = pl.pallas_call(
        _layer_body,
        out_shape=(jax.ShapeDtypeStruct((F, N), jnp.float32),
                   jax.ShapeDtypeStruct((N, F), jnp.float32)))
    last_tc = pl.pallas_call(
        _last_body,
        out_shape=jax.ShapeDtypeStruct((F, N), jnp.float32))
    fin_tc = pl.pallas_call(
        _make_fin(E),
        out_shape=jax.ShapeDtypeStruct((2, F), jnp.float32))

    GE = E // BE
    xe_open = pl.pallas_call(
        _xe_open_body,
        grid=(GE,),
        in_specs=[pl.BlockSpec((FE, BE), lambda t: (0, t)),
                  pl.BlockSpec((F, FE), lambda t: (0, 0))],
        out_specs=pl.BlockSpec((BE, F), lambda t: (t, 0)),
        out_shape=jax.ShapeDtypeStruct((E, F), jnp.float32))
    xe_close = pl.pallas_call(
        _xe_close_body,
        grid=(GE,),
        in_specs=[pl.BlockSpec((BE, F), lambda t: (t, 0)),
                  pl.BlockSpec((FE, F), lambda t: (0, 0))],
        out_specs=pl.BlockSpec((FE, BE), lambda t: (0, t)),
        out_shape=jax.ShapeDtypeStruct((FE, E), jnp.float32))

    xn1, yt = open_xn(KNopen, xn, KN[0])
    xet = xe_open(xe, KEopen)
    xn_out = None
    for l in range(nL):
        stats = pass_a(ii, jj, yt)
        mi = fin_tc(stats)
        xet, divp = pass_b(ii, jj, yt, xet, mi, zeros)
        if l < nL - 1:
            xn1, yt = layer_tc(divp, xn1, KE[l], KD[l], KN[l + 1])
        else:
            xn_out = last_tc(divp, xn1, KE[l], KD[l], KNclose)
    xe_out = xe_close(xet, KEclose)
    return (xn_out, xe_out)


# speedup vs baseline: 4.8384x; 1.5525x over previous
import functools

import jax
import jax.numpy as jnp
from jax import lax
from jax.experimental import pallas as pl
from jax.experimental.pallas import tpu as pltpu
from jax.experimental.pallas import tpu_sc as plsc

H = 0.1
EPS = 1e-3
F = 128
NW = 32
LANES = 16
C = 40
BE = 2560
NV = F // LANES


def _sc_mesh():
    return plsc.VectorSubcoreMesh(core_axis_name="c", subcore_axis_name="s")


def _make_pass_a(N, E):
    EW = E // NW
    NCH = EW // C
    SP = (NCH - 4) // 2

    @functools.partial(
        pl.kernel,
        mesh=_sc_mesh(),
        out_type=jax.ShapeDtypeStruct((NW, 2, F), jnp.float32),
        scratch_types=[
            pltpu.VMEM((C,), jnp.int32), pltpu.VMEM((C,), jnp.int32),
            pltpu.VMEM((C,), jnp.int32), pltpu.VMEM((C,), jnp.int32),
            pltpu.VMEM((C, F), jnp.float32), pltpu.VMEM((C, F), jnp.float32),
            pltpu.VMEM((C, F), jnp.float32), pltpu.VMEM((C, F), jnp.float32),
            pltpu.VMEM((2, F), jnp.float32),
            pltpu.SemaphoreType.DMA, pltpu.SemaphoreType.DMA,
            pltpu.SemaphoreType.DMA, pltpu.SemaphoreType.DMA,
        ],
    )
    def pass_a(ii, jj, yt, stats,
               iA, jA, iB, jB, riA, rjA, riB, rjB, st,
               semIA, semIB, semGA, semGB):
        wid = lax.axis_index("c") * 16 + lax.axis_index("s")
        base = wid * EW

        def gidx(x, iT, jT, sem):
            off = base + x * C
            pltpu.async_copy(ii.at[pl.ds(off, C)], iT, sem)
            pltpu.async_copy(jj.at[pl.ds(off, C)], jT, sem)

        def wait_gidx(iT, jT, sem):
            pltpu.make_async_copy(ii.at[pl.ds(base, C)], iT, sem).wait()
            pltpu.make_async_copy(jj.at[pl.ds(base, C)], jT, sem).wait()

        def gath(iT, jT, riT, rjT, sem):
            pltpu.async_copy(yt.at[iT], riT, sem)
            pltpu.async_copy(yt.at[jT], rjT, sem)

        def wait_gath(iT, jT, riT, rjT, sem):
            pltpu.make_async_copy(yt.at[iT], riT, sem).wait()
            pltpu.make_async_copy(yt.at[jT], rjT, sem).wait()

        def compute(riT, rjT, acc):
            def edge(e, a):
                out = list(a)
                for v in range(NV):
                    sl = pl.ds(v * LANES, LANES)
                    d = riT[e, sl] - rjT[e, sl]
                    out[v] = a[v] + d
                    out[NV + v] = a[NV + v] + d * d
                return tuple(out)
            return lax.fori_loop(0, C, edge, acc)

        A = (iA, jA, riA, rjA)
        B = (iB, jB, riB, rjB)

        pltpu.sync_copy(ii.at[pl.ds(base, C)], iA)
        pltpu.sync_copy(jj.at[pl.ds(base, C)], jA)
        gath(*A, semGA)
        gidx(1, iB, jB, semIB)

        zero = jnp.zeros((LANES,), jnp.float32)
        acc = tuple(zero for _ in range(2 * NV))

        wait_gath(*A, semGA)
        wait_gidx(iB, jB, semIB)
        gath(*B, semGB)
        gidx(2, iA, jA, semIA)
        acc = compute(riA, rjA, acc)
        wait_gath(*B, semGB)
        wait_gidx(iA, jA, semIA)
        gath(*A, semGA)
        gidx(3, iB, jB, semIB)
        acc = compute(riB, rjB, acc)

        def steady(x, P, Q, semI_P, semI_Q, semG_P, semG_Q, a):
            iP, jP, riP, rjP = P
            iQ, jQ, riQ, rjQ = Q
            wait_gath(*P, semG_P)
            wait_gidx(iQ, jQ, semI_Q)
            gath(*Q, semG_Q)
            gidx(x + 2, iP, jP, semI_P)
            return compute(riP, rjP, a)

        def body(c2, a):
            x = 2 + 2 * c2
            a = steady(x, A, B, semIA, semIB, semGA, semGB, a)
            a = steady(x + 1, B, A, semIB, semIA, semGB, semGA, a)
            return a

        acc = lax.fori_loop(0, SP, body, acc)

        wait_gath(*A, semGA)
        wait_gidx(iB, jB, semIB)
        gath(*B, semGB)
        acc = compute(riA, rjA, acc)
        wait_gath(*B, semGB)
        acc = compute(riB, rjB, acc)

        for v in range(NV):
            st[0, pl.ds(v * LANES, LANES)] = acc[v]
            st[1, pl.ds(v * LANES, LANES)] = acc[NV + v]
        pltpu.sync_copy(st, stats.at[wid])

    return pass_a


def _make_pass_b(N, E):
    EW = E // NW
    NCH = EW // C
    SP = (NCH - 4) // 2
    RPT = 1000

    @functools.partial(
        pl.kernel,
        mesh=_sc_mesh(),
        out_type=(
            jax.ShapeDtypeStruct((E, F), jnp.float32),
            jax.ShapeDtypeStruct((2, N, F), jnp.float32),
        ),
        scratch_types=[
            pltpu.VMEM((C,), jnp.int32), pltpu.VMEM((C,), jnp.int32),
            pltpu.VMEM((C,), jnp.int32), pltpu.VMEM((C,), jnp.int32),
            pltpu.VMEM((C,), jnp.int32), pltpu.VMEM((C,), jnp.int32),
            pltpu.VMEM((C,), jnp.int32), pltpu.VMEM((C,), jnp.int32),
            pltpu.VMEM((C, F), jnp.float32), pltpu.VMEM((C, F), jnp.float32),
            pltpu.VMEM((C, F), jnp.float32), pltpu.VMEM((C, F), jnp.float32),
            pltpu.VMEM((C, F), jnp.float32), pltpu.VMEM((C, F), jnp.float32),
            pltpu.VMEM((C, F), jnp.float32), pltpu.VMEM((C, F), jnp.float32),
            pltpu.VMEM((2, F), jnp.float32),
            pltpu.VMEM_SHARED((N, F), jnp.float32),
            pltpu.SemaphoreType.DMA, pltpu.SemaphoreType.DMA,
            pltpu.SemaphoreType.DMA, pltpu.SemaphoreType.DMA,
            pltpu.SemaphoreType.DMA, pltpu.SemaphoreType.DMA,
            pltpu.SemaphoreType.DMA, pltpu.SemaphoreType.DMA,
            pltpu.SemaphoreType.DMA, pltpu.SemaphoreType.DMA,
            pltpu.SemaphoreType.DMA, pltpu.SemaphoreType.DMA,
        ],
    )
    def pass_b(ii, jj, yt, xet, mi, zeros, xet_out, divp,
               iA, jA, iB, jB, siA, sjA, siB, sjB, riA, rjA, riB, rjB,
               xeA, xeB, negA, negB, mi_buf, shared,
               semIA, semIB, semXA, semXB, semGA, semGB,
               semLA, semLB, semWA, semWB, semSA, semSB):
        cid = lax.axis_index("c")
        sid = lax.axis_index("s")
        wid = cid * 16 + sid
        base = wid * EW

        pltpu.sync_copy(mi, mi_buf)

        @pl.when(sid == 0)
        def _():
            pltpu.sync_copy(zeros, shared)

        plsc.subcore_barrier()

        mean_vs = [mi_buf[0, pl.ds(v * LANES, LANES)] for v in range(NV)]
        inv_vs = [mi_buf[1, pl.ds(v * LANES, LANES)] for v in range(NV)]

        def gidx(x, iT, jT, sem):
            off = base + x * C
            pltpu.async_copy(ii.at[pl.ds(off, C)], iT, sem)
            pltpu.async_copy(jj.at[pl.ds(off, C)], jT, sem)

        def wait_gidx(iT, jT, sem):
            pltpu.make_async_copy(ii.at[pl.ds(base, C)], iT, sem).wait()
            pltpu.make_async_copy(jj.at[pl.ds(base, C)], jT, sem).wait()

        def gath(iT, jT, riT, rjT, sem):
            pltpu.async_copy(yt.at[iT], riT, sem)
            pltpu.async_copy(yt.at[jT], rjT, sem)

        def wait_gath(iT, jT, riT, rjT, sem):
            pltpu.make_async_copy(yt.at[iT], riT, sem).wait()
            pltpu.make_async_copy(yt.at[jT], rjT, sem).wait()

        def xeload(x, xeT, sem):
            off = base + x * C
            pltpu.async_copy(xet.at[pl.ds(off, C)], xeT, sem)

        def wait_xeload(xeT, sem):
            pltpu.make_async_copy(xet.at[pl.ds(base, C)], xeT, sem).wait()

        def stores(x, xeT, negT, siT, sjT, semW, semS):
            off = base + x * C
            pltpu.async_copy(xeT, xet_out.at[pl.ds(off, C)], semW)
            pltpu.async_copy(xeT, shared.at[siT], semS, add=True)
            pltpu.async_copy(negT, shared.at[sjT], semS, add=True)

        def wait_stores(xeT, negT, siT, sjT, semW, semS):
            pltpu.make_async_copy(xeT, xet_out.at[pl.ds(base, C)],
                                  semW).wait()
            pltpu.make_async_copy(xeT, shared.at[siT], semS).wait()
            pltpu.make_async_copy(negT, shared.at[sjT], semS).wait()

        def compute(riT, rjT, xeT, negT):
            def edge(e, cc):
                for v in range(NV):
                    sl = pl.ds(v * LANES, LANES)
                    r = jnp.maximum(
                        (riT[e, sl] - rjT[e, sl] - mean_vs[v]) * inv_vs[v],
                        0.0)
                    val = xeT[e, sl] + H * r
                    xeT[e, sl] = val
                    negT[e, sl] = -val
                return cc
            lax.fori_loop(0, C, edge, 0)

        pltpu.sync_copy(ii.at[pl.ds(base, C)], iA)
        pltpu.sync_copy(jj.at[pl.ds(base, C)], jA)
        pltpu.sync_copy(ii.at[pl.ds(base, C)], siA)
        pltpu.sync_copy(jj.at[pl.ds(base, C)], sjA)
        gath(iA, jA, riA, rjA, semGA)
        xeload(0, xeA, semLA)
        gidx(1, iB, jB, semIB)
        gidx(1, siB, sjB, semXB)

        wait_gath(iA, jA, riA, rjA, semGA)
        wait_xeload(xeA, semLA)
        wait_gidx(iB, jB, semIB)
        gath(iB, jB, riB, rjB, semGB)
        xeload(1, xeB, semLB)
        gidx(2, iA, jA, semIA)
        compute(riA, rjA, xeA, negA)
        stores(0, xeA, negA, siA, sjA, semWA, semSA)

        wait_gath(iB, jB, riB, rjB, semGB)
        wait_xeload(xeB, semLB)
        wait_gidx(iA, jA, semIA)
        wait_stores(xeA, negA, siA, sjA, semWA, semSA)
        gidx(2, siA, sjA, semXA)
        gath(iA, jA, riA, rjA, semGA)
        xeload(2, xeA, semLA)
        gidx(3, iB, jB, semIB)
        wait_gidx(siB, sjB, semXB)
        compute(riB, rjB, xeB, negB)
        stores(1, xeB, negB, siB, sjB, semWB, semSB)

        def steady(x, iP, jP, siP, sjP, riP, rjP, xeP, negP,
                   iQ, jQ, siQ, sjQ, riQ, rjQ, xeQ, negQ,
                   semI_P, semI_Q, semX_P, semX_Q, semG_P, semG_Q,
                   semL_P, semL_Q, semW_P, semW_Q, semS_P, semS_Q):
            wait_gath(iP, jP, riP, rjP, semG_P)
            wait_xeload(xeP, semL_P)
            wait_gidx(iQ, jQ, semI_Q)
            wait_stores(xeQ, negQ, siQ, sjQ, semW_Q, semS_Q)
            gidx(x + 1, siQ, sjQ, semX_Q)
            gath(iQ, jQ, riQ, rjQ, semG_Q)
            xeload(x + 1, xeQ, semL_Q)
            gidx(x + 2, iP, jP, semI_P)
            wait_gidx(siP, sjP, semX_P)
            compute(riP, rjP, xeP, negP)
            stores(x, xeP, negP, siP, sjP, semW_P, semS_P)

        def body(c2, cc):
            x = 2 + 2 * c2
            steady(x, iA, jA, siA, sjA, riA, rjA, xeA, negA,
                   iB, jB, siB, sjB, riB, rjB, xeB, negB,
                   semIA, semIB, semXA, semXB, semGA, semGB,
                   semLA, semLB, semWA, semWB, semSA, semSB)
            steady(x + 1, iB, jB, siB, sjB, riB, rjB, xeB, negB,
                   iA, jA, siA, sjA, riA, rjA, xeA, negA,
                   semIB, semIA, semXB, semXA, semGB, semGA,
                   semLB, semLA, semWB, semWA, semSB, semSA)
            return cc

        lax.fori_loop(0, SP, body, 0)

        wait_gath(iA, jA, riA, rjA, semGA)
        wait_xeload(xeA, semLA)
        wait_gidx(iB, jB, semIB)
        wait_stores(xeB, negB, siB, sjB, semWB, semSB)
        gidx(NCH - 1, siB, sjB, semXB)
        gath(iB, jB, riB, rjB, semGB)
        xeload(NCH - 1, xeB, semLB)
        wait_gidx(siA, sjA, semXA)
        compute(riA, rjA, xeA, negA)
        stores(NCH - 2, xeA, negA, siA, sjA, semWA, semSA)

        wait_gath(iB, jB, riB, rjB, semGB)
        wait_xeload(xeB, semLB)
        wait_stores(xeA, negA, siA, sjA, semWA, semSA)
        wait_gidx(siB, sjB, semXB)
        compute(riB, rjB, xeB, negB)
        stores(NCH - 1, xeB, negB, siB, sjB, semWB, semSB)

        wait_stores(xeB, negB, siB, sjB, semWB, semSB)

        plsc.subcore_barrier()

        @pl.when(sid < N // RPT)
        def _():
            r0 = sid * RPT
            pltpu.sync_copy(shared.at[pl.ds(r0, RPT)],
                            divp.at[cid, pl.ds(r0, RPT)])

    return pass_b


def _tvn_rows(X):
    Xc = X - jnp.mean(X, axis=1, keepdims=True)
    return Xc / jnp.sqrt(jnp.sum(Xc * Xc, axis=1, keepdims=True) + EPS)


def _dot(a, b, dims):
    return lax.dot_general(a, b, (dims, ((), ())),
                           preferred_element_type=jnp.float32)


def _open_xn_body(kno, xn, kn0, xn1_out, yt0_out):
    xn1 = _dot(kno[...], xn[...], ((1,), (0,)))
    xn1_out[...] = xn1
    yt0_out[...] = _dot(xn1, kn0[...], ((0,), (1,)))


def _layer_body(divp, xn, ke, kd, knn, xn_out, yt_out):
    P = divp[0] + divp[1]
    Bi = jnp.maximum(_tvn_rows(_dot(ke[...], P, ((1,), (1,)))), 0.0)
    Ci = jnp.maximum(_tvn_rows(_dot(kd[...], xn[...], ((1,), (0,)))), 0.0)
    xn_new = xn[...] - H * (Bi + Ci)
    xn_out[...] = xn_new
    yt_out[...] = _dot(xn_new, knn[...], ((0,), (1,)))


def _last_body(divp, xn, ke, kd, knc, xn_out):
    P = divp[0] + divp[1]
    Bi = jnp.maximum(_tvn_rows(_dot(ke[...], P, ((1,), (1,)))), 0.0)
    Ci = jnp.maximum(_tvn_rows(_dot(kd[...], xn[...], ((1,), (0,)))), 0.0)
    xn_new = xn[...] - H * (Bi + Ci)
    xn_out[...] = _dot(knc[...], xn_new, ((1,), (0,)))


def _make_fin(E):
    def _fin_body(stats, out):
        T = jnp.sum(stats[...], axis=0)
        m = T[0:1] / float(E)
        q = T[1:2]
        inv = lax.rsqrt(q - float(E) * m * m + EPS)
        out[...] = jnp.concatenate([m, inv], axis=0)
    return _fin_body


def _xe_open_body(xe, keo, out):
    out[...] = _dot(xe[...], keo[...], ((0,), (1,)))


def _xe_close_body(xet, kec, out):
    out[...] = _dot(kec[...], xet[...], ((1,), (1,)))


def kernel(xn, xe, edge_index, KNopen, KEopen, KNclose, KEclose, KN, KE, KD):
    N = xn.shape[1]
    E = xe.shape[1]
    FE = xe.shape[0]
    nL = KN.shape[0]
    ii = edge_index[0]
    jj = edge_index[1]
    zeros = jnp.zeros((N, F), jnp.float32)

    pass_a = _make_pass_a(N, E)
    pass_b = _make_pass_b(N, E)

    open_xn = pl.pallas_call(
        _open_xn_body,
        out_shape=(jax.ShapeDtypeStruct((F, N), jnp.float32),
                   jax.ShapeDtypeStruct((N, F), jnp.float32)))
    layer_tc = pl.pallas_call(
        _layer_body,
        out_shape=(jax.ShapeDtypeStruct((F, N), jnp.float32),
                   jax.ShapeDtypeStruct((N, F), jnp.float32)))
    last_tc = pl.pallas_call(
        _last_body,
        out_shape=jax.ShapeDtypeStruct((F, N), jnp.float32))
    fin_tc = pl.pallas_call(
        _make_fin(E),
        out_shape=jax.ShapeDtypeStruct((2, F), jnp.float32))

    GE = E // BE
    xe_open = pl.pallas_call(
        _xe_open_body,
        grid=(GE,),
        in_specs=[pl.BlockSpec((FE, BE), lambda t: (0, t)),
                  pl.BlockSpec((F, FE), lambda t: (0, 0))],
        out_specs=pl.BlockSpec((BE, F), lambda t: (t, 0)),
        out_shape=jax.ShapeDtypeStruct((E, F), jnp.float32))
    xe_close = pl.pallas_call(
        _xe_close_body,
        grid=(GE,),
        in_specs=[pl.BlockSpec((BE, F), lambda t: (t, 0)),
                  pl.BlockSpec((FE, F), lambda t: (0, 0))],
        out_specs=pl.BlockSpec((FE, BE), lambda t: (0, t)),
        out_shape=jax.ShapeDtypeStruct((FE, E), jnp.float32))

    xn1, yt = open_xn(KNopen, xn, KN[0])
    xet = xe_open(xe, KEopen)
    xn_out = None
    for l in range(nL):
        stats = pass_a(ii, jj, yt)
        mi = fin_tc(stats)
        xet, divp = pass_b(ii, jj, yt, xet, mi, zeros)
        if l < nL - 1:
            xn1, yt = layer_tc(divp, xn1, KE[l], KD[l], KN[l + 1])
        else:
            xn_out = last_tc(divp, xn1, KE[l], KD[l], KNclose)
    xe_out = xe_close(xet, KEclose)
    return (xn_out, xe_out)
